# baseline (speedup 1.0000x reference)
import functools

import jax
import jax.numpy as jnp
from jax import lax
from jax.experimental import pallas as pl
from jax.experimental.pallas import tpu as pltpu
from jax.experimental.pallas import tpu_sc as plsc

E = 8
TOPK = 2
BLK = 512
BT = 512
IC = 1024

_SC = plsc.get_sparse_core_info()
NW = _SC.num_cores * _SC.num_subcores


def _router_body(x_ref, g_ref, b_ref, wr_ref, br_ref, ew_ref, ei_ref):
    x = x_ref[...]
    mu = jnp.mean(x, axis=1, keepdims=True)
    var = jnp.mean((x - mu) ** 2, axis=1, keepdims=True)
    norm = (x - mu) * jax.lax.rsqrt(var + 1e-5) * g_ref[...] + b_ref[...]
    logits = jnp.dot(norm, wr_ref[...], preferred_element_type=jnp.float32)
    logits = logits + br_ref[...]
    iota = jax.lax.broadcasted_iota(jnp.int32, logits.shape, 1)
    m1 = jnp.max(logits, axis=1, keepdims=True)
    a1 = jnp.min(jnp.where(logits == m1, iota, E), axis=1, keepdims=True)
    masked = jnp.where(iota == a1, -jnp.inf, logits)
    m2 = jnp.max(masked, axis=1, keepdims=True)
    a2 = jnp.min(jnp.where(masked == m2, iota, E), axis=1, keepdims=True)
    t = jnp.exp(m2 - m1)
    w1 = 1.0 / (1.0 + t)
    ew_ref[...] = jnp.concatenate([w1, 1.0 - w1], axis=1)
    ei_ref[...] = jnp.concatenate([a1, a2], axis=1)


def _ffn_body(be_ref, na_ref, xs_ref, wu_ref, bu_ref, wd_ref, bd_ref,
              ws_ref, ys_ref):
    b = pl.program_id(0)

    @pl.when(b < na_ref[0])
    def _():
        inter = wu_ref.shape[2]
        xb = xs_ref[...]
        acc = jnp.zeros((BLK, xs_ref.shape[1]), jnp.float32)
        for c in range(inter // IC):
            sl = pl.ds(c * IC, IC)
            h = jnp.dot(xb, wu_ref[0, :, sl],
                        preferred_element_type=jnp.float32)
            h = h + bu_ref[0, 0, sl]
            h = 0.5 * h * (1.0 + jax.lax.erf(h * (2.0 ** -0.5)))
            acc = acc + jnp.dot(h.astype(jnp.bfloat16), wd_ref[0, sl, :],
                                preferred_element_type=jnp.float32)
        y = (acc + bd_ref[0, 0, :]) * ws_ref[0, 0, :][:, None]
        ys_ref[...] = y


def _sc_gather(x, src, cap, h):
    rows_per_w = cap // NW
    ch = 48
    nbuf = 4
    nch = rows_per_w // ch
    assert rows_per_w % ch == 0

    @functools.partial(
        pl.kernel,
        mesh=plsc.VectorSubcoreMesh(core_axis_name="c", subcore_axis_name="s"),
        out_type=jax.ShapeDtypeStruct((cap, h), jnp.int32),
        scratch_types=(
            [pltpu.VMEM((rows_per_w,), jnp.int32)]
            + [pltpu.VMEM((ch, h), jnp.int32) for _ in range(nbuf)]
            + [pltpu.SemaphoreType.DMA for _ in range(nbuf)]
            + [pltpu.SemaphoreType.DMA for _ in range(nbuf)]
        ),
    )
    def body(x_hbm, src_hbm, xs_hbm, idx_v, *rest):
        bufs = rest[:nbuf]
        gsems = rest[nbuf:2 * nbuf]
        wsems = rest[2 * nbuf:3 * nbuf]
        wid = lax.axis_index("s") * _SC.num_cores + lax.axis_index("c")
        base = wid * rows_per_w
        pltpu.sync_copy(src_hbm.at[pl.ds(pl.multiple_of(base, 8),
                                         rows_per_w)], idx_v)
        gc = [None] * nch
        wc = [None] * nch
        for c in range(min(nbuf, nch)):
            gc[c] = pltpu.async_copy(
                x_hbm.at[idx_v.at[pl.ds(c * ch, ch)]], bufs[c], gsems[c])
        for c in range(nch):
            b = c % nbuf
            gc[c].wait()
            off = pl.multiple_of(base + c * ch, 8)
            wc[c] = pltpu.async_copy(bufs[b], xs_hbm.at[pl.ds(off, ch)],
                                     wsems[b])
            n = c + nbuf
            if n < nch:
                wc[c].wait()
                gc[n] = pltpu.async_copy(
                    x_hbm.at[idx_v.at[pl.ds(n * ch, ch)]], bufs[b], gsems[b])
        for c in range(max(nch - nbuf, 0), nch):
            wc[c].wait()

    return body(x, src)


def _sc_combine(ys, p0, p1, t, h):
    t_per_w = t // NW
    ch = 32
    assert t_per_w % ch == 0

    @functools.partial(
        pl.kernel,
        mesh=plsc.VectorSubcoreMesh(core_axis_name="c", subcore_axis_name="s"),
        out_type=jax.ShapeDtypeStruct((t, h), jnp.float32),
        scratch_types=[
            pltpu.VMEM((ch,), jnp.int32),
            pltpu.VMEM((ch,), jnp.int32),
            pltpu.VMEM((ch, h), jnp.float32),
            pltpu.VMEM((ch, h), jnp.float32),
            pltpu.SemaphoreType.DMA,
            pltpu.SemaphoreType.DMA,
        ],
    )
    def body(ys_hbm, p0_hbm, p1_hbm, out_hbm, i0, i1, r0, r1, s0, s1):
        wid = lax.axis_index("s") * _SC.num_cores + lax.axis_index("c")
        base = wid * t_per_w
        for c in range(t_per_w // ch):
            off = pl.multiple_of(base + c * ch, 8)
            pltpu.sync_copy(p0_hbm.at[pl.ds(off, ch)], i0)
            pltpu.sync_copy(p1_hbm.at[pl.ds(off, ch)], i1)
            cp0 = pltpu.async_copy(ys_hbm.at[i0], r0, s0)
            cp1 = pltpu.async_copy(ys_hbm.at[i1], r1, s1)
            cp0.wait()
            cp1.wait()

            def add_row(row, _):
                def add_vec(j, _):
                    sl = pl.ds(j * 16, 16)
                    r0[row, sl] = r0[row, sl] + r1[row, sl]
                    return 0
                return lax.fori_loop(0, h // 16, add_vec, 0, unroll=4)

            lax.fori_loop(0, ch, add_row, 0)
            pltpu.sync_copy(r0, out_hbm.at[pl.ds(off, ch)])

    return body(ys, p0, p1)


def kernel(hidden_states, ln_g, ln_b, Wr, br, Wu, bu, Wd, bd):
    Bsz, Sq, H = hidden_states.shape
    T = Bsz * Sq
    A = T * TOPK
    NB = A // BLK + E
    CAP = NB * BLK
    inter = Wu.shape[2]
    x = hidden_states.reshape(T, H)

    ew, ei = pl.pallas_call(
        _router_body,
        grid=(T // BT,),
        in_specs=[
            pl.BlockSpec((BT, H), lambda i: (i, 0)),
            pl.BlockSpec((H,), lambda i: (0,)),
            pl.BlockSpec((H,), lambda i: (0,)),
            pl.BlockSpec((H, E), lambda i: (0, 0)),
            pl.BlockSpec((E,), lambda i: (0,)),
        ],
        out_specs=[
            pl.BlockSpec((BT, TOPK), lambda i: (i, 0)),
            pl.BlockSpec((BT, TOPK), lambda i: (i, 0)),
        ],
        out_shape=[
            jax.ShapeDtypeStruct((T, TOPK), jnp.float32),
            jax.ShapeDtypeStruct((T, TOPK), jnp.int32),
        ],
    )(x, ln_g, ln_b, Wr, br)

    ef = ei.reshape(A)
    wf = ew.reshape(A)
    onehot = (ef[:, None] == jnp.arange(E, dtype=jnp.int32)[None, :]
              ).astype(jnp.int32)
    rank = jnp.take_along_axis(jnp.cumsum(onehot, axis=0) - onehot,
                               ef[:, None], axis=1)[:, 0]
    counts = jnp.sum(onehot, axis=0)
    padded = ((counts + BLK - 1) // BLK) * BLK
    cum = jnp.cumsum(padded)
    pos = (cum - padded)[ef] + rank
    tokens = jnp.arange(A, dtype=jnp.int32) // TOPK
    src = jnp.zeros((CAP,), jnp.int32).at[pos].set(tokens)
    ws = jnp.zeros((CAP,), jnp.float32).at[pos].set(wf)
    be = jnp.minimum(
        jnp.searchsorted(cum, jnp.arange(NB, dtype=jnp.int32) * BLK,
                         side="right"),
        E - 1).astype(jnp.int32)
    na = (cum[-1] // BLK).astype(jnp.int32).reshape(1)

    x_i32 = lax.bitcast_convert_type(
        x.astype(jnp.bfloat16).reshape(T, H // 2, 2), jnp.int32)
    xs = lax.bitcast_convert_type(
        _sc_gather(x_i32, src, CAP, H // 2), jnp.bfloat16).reshape(CAP, H)

    grid_spec = pltpu.PrefetchScalarGridSpec(
        num_scalar_prefetch=2,
        grid=(NB,),
        in_specs=[
            pl.BlockSpec((BLK, H), lambda b, be, na: (b, 0)),
            pl.BlockSpec((1, H, inter), lambda b, be, na: (be[b], 0, 0)),
            pl.BlockSpec((1, 1, inter), lambda b, be, na: (be[b], 0, 0)),
            pl.BlockSpec((1, inter, H), lambda b, be, na: (be[b], 0, 0)),
            pl.BlockSpec((1, 1, H), lambda b, be, na: (be[b], 0, 0)),
            pl.BlockSpec((1, 1, BLK), lambda b, be, na: (b, 0, 0)),
        ],
        out_specs=pl.BlockSpec((BLK, H), lambda b, be, na: (b, 0)),
    )
    ys = pl.pallas_call(
        _ffn_body,
        grid_spec=grid_spec,
        out_shape=jax.ShapeDtypeStruct((CAP, H), jnp.float32),
    )(be, na, xs, Wu.astype(jnp.bfloat16), bu.reshape(E, 1, inter),
      Wd.astype(jnp.bfloat16), bd.reshape(E, 1, H), ws.reshape(NB, 1, BLK))

    out = _sc_combine(ys, pos[0::TOPK], pos[1::TOPK], T, H)
    return out.reshape(Bsz, Sq, H)


# speedup vs baseline: 1.9953x; 1.9953x over previous
import functools

import jax
import jax.numpy as jnp
from jax import lax
from jax.experimental import pallas as pl
from jax.experimental.pallas import tpu as pltpu
from jax.experimental.pallas import tpu_sc as plsc

E = 8
TOPK = 2
BLK = 512
BT = 512
IC = 1024

_SC = plsc.get_sparse_core_info()
NW = _SC.num_cores * _SC.num_subcores


def _router_body(x_ref, g_ref, b_ref, wr_ref, br_ref, ew_ref, ei_ref):
    x = x_ref[...]
    mu = jnp.mean(x, axis=1, keepdims=True)
    var = jnp.mean((x - mu) ** 2, axis=1, keepdims=True)
    norm = (x - mu) * jax.lax.rsqrt(var + 1e-5) * g_ref[...] + b_ref[...]
    logits = jnp.dot(norm, wr_ref[...], preferred_element_type=jnp.float32)
    logits = logits + br_ref[...]
    iota = jax.lax.broadcasted_iota(jnp.int32, logits.shape, 1)
    m1 = jnp.max(logits, axis=1, keepdims=True)
    a1 = jnp.min(jnp.where(logits == m1, iota, E), axis=1, keepdims=True)
    masked = jnp.where(iota == a1, -jnp.inf, logits)
    m2 = jnp.max(masked, axis=1, keepdims=True)
    a2 = jnp.min(jnp.where(masked == m2, iota, E), axis=1, keepdims=True)
    t = jnp.exp(m2 - m1)
    w1 = 1.0 / (1.0 + t)
    ew_ref[...] = jnp.concatenate([w1, 1.0 - w1], axis=1)
    ei_ref[...] = jnp.concatenate([a1, a2], axis=1)


def _ffn_body(be_ref, na_ref, xs_ref, wu_ref, bu_ref, wd_ref, bd_ref,
              ws_ref, ys_ref):
    b = pl.program_id(0)

    @pl.when(b < na_ref[0])
    def _():
        inter = wu_ref.shape[2]
        xb = xs_ref[...].astype(jnp.bfloat16)
        acc = jnp.zeros((BLK, xs_ref.shape[1]), jnp.float32)
        for c in range(inter // IC):
            sl = pl.ds(c * IC, IC)
            h = jnp.dot(xb, wu_ref[0, :, sl],
                        preferred_element_type=jnp.float32)
            h = h + bu_ref[0, 0, sl]
            h = 0.5 * h * (1.0 + jax.lax.erf(h * (2.0 ** -0.5)))
            acc = acc + jnp.dot(h.astype(jnp.bfloat16), wd_ref[0, sl, :],
                                preferred_element_type=jnp.float32)
        y = (acc + bd_ref[0, 0, :]) * ws_ref[0, 0, :][:, None]
        ys_ref[...] = y


def _sc_gather(x, src, cap, h):
    t = x.shape[0]
    qc = 128
    nph = h // (qc * _SC.num_cores)
    rows_per_w = cap // _SC.num_subcores
    stage_rows = t // _SC.num_subcores
    ch = 64
    nbuf = 4
    nch = rows_per_w // ch
    assert rows_per_w % ch == 0 and t % _SC.num_subcores == 0

    @functools.partial(
        pl.kernel,
        mesh=plsc.VectorSubcoreMesh(core_axis_name="c", subcore_axis_name="s"),
        out_type=jax.ShapeDtypeStruct((cap, h), jnp.float32),
        scratch_types=(
            [pltpu.VMEM((rows_per_w,), jnp.int32),
             pltpu.VMEM_SHARED((t, qc), jnp.float32)]
            + [pltpu.VMEM((ch, qc), jnp.float32) for _ in range(nbuf)]
            + [pltpu.SemaphoreType.DMA for _ in range(nbuf)]
            + [pltpu.SemaphoreType.DMA for _ in range(nbuf)]
        ),
    )
    def body(x_hbm, src_hbm, xs_hbm, idx_v, shared, *rest):
        bufs = rest[:nbuf]
        gsems = rest[nbuf:2 * nbuf]
        wsems = rest[2 * nbuf:3 * nbuf]
        core = lax.axis_index("c")
        sid = lax.axis_index("s")
        base = sid * rows_per_w
        pltpu.sync_copy(src_hbm.at[pl.ds(pl.multiple_of(base, 8),
                                         rows_per_w)], idx_v)
        for p in range(nph):
            col0 = p * (qc * _SC.num_cores) + core * qc
            srow = sid * stage_rows
            pltpu.sync_copy(
                x_hbm.at[pl.ds(srow, stage_rows), pl.ds(col0, qc)],
                shared.at[pl.ds(srow, stage_rows)])
            plsc.subcore_barrier()
            gc = [None] * nch
            wc = [None] * nch
            for c in range(min(nbuf, nch)):
                gc[c] = pltpu.async_copy(
                    shared.at[idx_v.at[pl.ds(c * ch, ch)]], bufs[c], gsems[c])
            for c in range(nch):
                b = c % nbuf
                gc[c].wait()
                wc[c] = pltpu.async_copy(
                    bufs[b],
                    xs_hbm.at[pl.ds(base + c * ch, ch), pl.ds(col0, qc)],
                    wsems[b])
                n = c + nbuf
                if n < nch:
                    wc[c].wait()
                    gc[n] = pltpu.async_copy(
                        shared.at[idx_v.at[pl.ds(n * ch, ch)]], bufs[b],
                        gsems[b])
            for c in range(max(nch - nbuf, 0), nch):
                wc[c].wait()
            if p + 1 < nph:
                plsc.subcore_barrier()

    return body(x, src)


def _sc_combine(ys, p0, p1, t, h):
    t_per_w = t // NW
    ch = 32
    assert t_per_w % ch == 0

    @functools.partial(
        pl.kernel,
        mesh=plsc.VectorSubcoreMesh(core_axis_name="c", subcore_axis_name="s"),
        out_type=jax.ShapeDtypeStruct((t, h), jnp.float32),
        scratch_types=[
            pltpu.VMEM((ch,), jnp.int32),
            pltpu.VMEM((ch,), jnp.int32),
            pltpu.VMEM((ch, h), jnp.float32),
            pltpu.VMEM((ch, h), jnp.float32),
            pltpu.SemaphoreType.DMA,
            pltpu.SemaphoreType.DMA,
        ],
    )
    def body(ys_hbm, p0_hbm, p1_hbm, out_hbm, i0, i1, r0, r1, s0, s1):
        wid = lax.axis_index("s") * _SC.num_cores + lax.axis_index("c")
        base = wid * t_per_w
        for c in range(t_per_w // ch):
            off = pl.multiple_of(base + c * ch, 8)
            pltpu.sync_copy(p0_hbm.at[pl.ds(off, ch)], i0)
            pltpu.sync_copy(p1_hbm.at[pl.ds(off, ch)], i1)
            cp0 = pltpu.async_copy(ys_hbm.at[i0], r0, s0)
            cp1 = pltpu.async_copy(ys_hbm.at[i1], r1, s1)
            cp0.wait()
            cp1.wait()

            def add_row(row, _):
                def add_vec(j, _):
                    sl = pl.ds(j * 16, 16)
                    r0[row, sl] = r0[row, sl] + r1[row, sl]
                    return 0
                return lax.fori_loop(0, h // 16, add_vec, 0, unroll=4)

            lax.fori_loop(0, ch, add_row, 0)
            pltpu.sync_copy(r0, out_hbm.at[pl.ds(off, ch)])

    return body(ys, p0, p1)


def kernel(hidden_states, ln_g, ln_b, Wr, br, Wu, bu, Wd, bd):
    Bsz, Sq, H = hidden_states.shape
    T = Bsz * Sq
    A = T * TOPK
    NB = A // BLK + E
    CAP = NB * BLK
    inter = Wu.shape[2]
    x = hidden_states.reshape(T, H)

    ew, ei = pl.pallas_call(
        _router_body,
        grid=(T // BT,),
        in_specs=[
            pl.BlockSpec((BT, H), lambda i: (i, 0)),
            pl.BlockSpec((H,), lambda i: (0,)),
            pl.BlockSpec((H,), lambda i: (0,)),
            pl.BlockSpec((H, E), lambda i: (0, 0)),
            pl.BlockSpec((E,), lambda i: (0,)),
        ],
        out_specs=[
            pl.BlockSpec((BT, TOPK), lambda i: (i, 0)),
            pl.BlockSpec((BT, TOPK), lambda i: (i, 0)),
        ],
        out_shape=[
            jax.ShapeDtypeStruct((T, TOPK), jnp.float32),
            jax.ShapeDtypeStruct((T, TOPK), jnp.int32),
        ],
    )(x, ln_g, ln_b, Wr, br)

    ef = ei.reshape(A)
    wf = ew.reshape(A)
    onehot = (ef[:, None] == jnp.arange(E, dtype=jnp.int32)[None, :]
              ).astype(jnp.int32)
    rank = jnp.take_along_axis(jnp.cumsum(onehot, axis=0) - onehot,
                               ef[:, None], axis=1)[:, 0]
    counts = jnp.sum(onehot, axis=0)
    padded = ((counts + BLK - 1) // BLK) * BLK
    cum = jnp.cumsum(padded)
    pos = (cum - padded)[ef] + rank
    tokens = jnp.arange(A, dtype=jnp.int32) // TOPK
    src = jnp.zeros((CAP,), jnp.int32).at[pos].set(tokens)
    ws = jnp.zeros((CAP,), jnp.float32).at[pos].set(wf)
    be = jnp.minimum(
        jnp.searchsorted(cum, jnp.arange(NB, dtype=jnp.int32) * BLK,
                         side="right"),
        E - 1).astype(jnp.int32)
    na = (cum[-1] // BLK).astype(jnp.int32).reshape(1)

    xs = _sc_gather(x, src, CAP, H)

    grid_spec = pltpu.PrefetchScalarGridSpec(
        num_scalar_prefetch=2,
        grid=(NB,),
        in_specs=[
            pl.BlockSpec((BLK, H), lambda b, be, na: (b, 0)),
            pl.BlockSpec((1, H, inter), lambda b, be, na: (be[b], 0, 0)),
            pl.BlockSpec((1, 1, inter), lambda b, be, na: (be[b], 0, 0)),
            pl.BlockSpec((1, inter, H), lambda b, be, na: (be[b], 0, 0)),
            pl.BlockSpec((1, 1, H), lambda b, be, na: (be[b], 0, 0)),
            pl.BlockSpec((1, 1, BLK), lambda b, be, na: (b, 0, 0)),
        ],
        out_specs=pl.BlockSpec((BLK, H), lambda b, be, na: (b, 0)),
    )
    ys = pl.pallas_call(
        _ffn_body,
        grid_spec=grid_spec,
        out_shape=jax.ShapeDtypeStruct((CAP, H), jnp.float32),
    )(be, na, xs, Wu.astype(jnp.bfloat16), bu.reshape(E, 1, inter),
      Wd.astype(jnp.bfloat16), bd.reshape(E, 1, H), ws.reshape(NB, 1, BLK))

    out = _sc_combine(ys, pos[0::TOPK], pos[1::TOPK], T, H)
    return out.reshape(Bsz, Sq, H)
